# Initial kernel scaffold; baseline (speedup 1.0000x reference)
import jax
import jax.numpy as jnp
from jax.experimental import pallas as pl


def kernel(word_table, ent_table, word_ids, entity_ids):
    raise NotImplementedError("write your pallas kernel here")


# speedup vs baseline: 1.1642x; 1.1642x over previous
import functools

import jax
import jax.numpy as jnp
from jax import lax
from jax.experimental import pallas as pl
from jax.experimental.pallas import tpu as pltpu
from jax.experimental.pallas import tpu_sc as plsc

ENT_COUNT = 1000000
WORD_VOCAB = 100000
VEC = 64
ROWS = 204800
JPB = 200
NC, NS, L = 2, 16, 16
NW = NC * NS
RPW = ROWS // NW
EPW = 32
NBLK = EPW // L
RPB = L * JPB
PIECE = 128
PPW = RPW // PIECE
PPB = PPW // NBLK
JPP = PIECE // L


def _rsqrt(x):
    i = plsc.bitcast(x, jnp.int32)
    y = plsc.bitcast(jnp.int32(0x5F3759DF) - lax.shift_right_logical(i, 1),
                     jnp.float32)
    for _ in range(3):
        y = y * (1.5 - 0.5 * x * y * y)
    return y


@functools.partial(
    pl.kernel,
    out_type=jax.ShapeDtypeStruct((ROWS,), jnp.float32),
    mesh=plsc.VectorSubcoreMesh(core_axis_name="c", subcore_axis_name="s"),
    scratch_types=[
        pltpu.VMEM((RPW,), jnp.int32),
        pltpu.VMEM((PPW, PIECE), jnp.int32),
        pltpu.VMEM((NBLK, L), jnp.int32),
        pltpu.VMEM((L, VEC), jnp.float32),
        pltpu.VMEM((NBLK * VEC * L,), jnp.float32),
        pltpu.VMEM((PIECE, VEC), jnp.float32),
        pltpu.VMEM((PIECE, VEC), jnp.float32),
        pltpu.VMEM((RPW,), jnp.float32),
        pltpu.SemaphoreType.DMA,
        pltpu.SemaphoreType.DMA,
        pltpu.SemaphoreType.DMA,
    ],
    compiler_params=pltpu.CompilerParams(needs_layout_passes=False,
                                         use_tc_tiling_on_sc=False),
)
def _sc_cosine(wt_hbm, et_hbm, wid_hbm, eid_hbm, out_hbm,
               ids_v, perm_v, ents_v, erow_v, E_v, buf0, buf1, out_v,
               sem0, sem1, sem_m):
    wid = lax.axis_index("s") * NC + lax.axis_index("c")
    iota = lax.iota(jnp.int32, L)
    zf = jnp.zeros((L,), jnp.float32)

    pltpu.sync_copy(wid_hbm.at[pl.ds(wid * RPW, RPW)], ids_v)
    for t in range(NBLK):
        pltpu.sync_copy(eid_hbm.at[pl.ds(wid * EPW + t * L, L)], ents_v.at[t])

    def _perm(j2, carry):
        t = j2 // JPB
        j = j2 - t * JPB
        v = plsc.load_gather(ids_v, [iota * JPB + (t * RPB + j)])
        row = jnp.zeros((L,), jnp.int32) + (j2 // JPP)
        col = (j2 % JPP) * L + iota
        plsc.store_scatter(perm_v, [row, col], v)
        return carry

    lax.fori_loop(0, NBLK * JPB, _perm, 0)

    for t in range(NBLK):
        pltpu.async_copy(et_hbm.at[ents_v.at[t]], erow_v, sem_m).wait()

        def _esq(k, sq):
            g = plsc.load_gather(erow_v, [iota, jnp.zeros((L,), jnp.int32) + k])
            return sq + g * g

        sq = lax.fori_loop(0, VEC, _esq, zf)
        rs = _rsqrt(jnp.maximum(sq, 1e-30))
        n = sq * rs
        scale = jnp.where(sq > 1.0, 1.0 / (n + 1e-7), jnp.ones((L,), jnp.float32))

        def _escale(k, carry):
            g = plsc.load_gather(erow_v, [iota, jnp.zeros((L,), jnp.int32) + k])
            E_v[pl.ds(t * VEC * L + k * L, L)] = g * carry
            return carry

        lax.fori_loop(0, VEC, _escale, scale)

    bufs = (buf0, buf1)
    sems = (sem0, sem1)
    for b in range(2):
        pltpu.async_copy(wt_hbm.at[perm_v.at[b]], bufs[b], sems[b])

    def _compute_piece(m, buf):
        t = m // PPB
        mm = m - t * PPB

        def _jbody(jj, carry):
            j = mm * JPP + jj

            def _kbody(k, acc):
                sq, dot = acc
                v = plsc.load_gather(
                    buf, [iota + jj * L, jnp.zeros((L,), jnp.int32) + k])
                e = E_v[pl.ds(t * VEC * L + k * L, L)]
                return (sq + v * v, dot + v * e)

            sq, dot = (zf, zf)
            for k in range(VEC):
                sq, dot = _kbody(k, (sq, dot))
            val = dot * _rsqrt(jnp.maximum(sq, 1e-30))
            plsc.store_scatter(out_v, [iota * JPB + (t * RPB + j)], val)
            return carry

        lax.fori_loop(0, JPP, _jbody, 0)

    def _main(m2, carry):
        for b in range(2):
            m = m2 * 2 + b
            pltpu.make_async_copy(wt_hbm.at[perm_v.at[m]], bufs[b],
                                  sems[b]).wait()
            _compute_piece(m, bufs[b])
            nxt = m + 2

            @pl.when(nxt < PPW)
            def _fire():
                pltpu.async_copy(wt_hbm.at[perm_v.at[nxt]], bufs[b], sems[b])

        return carry

    lax.fori_loop(0, PPW // 2, _main, 0)

    pltpu.sync_copy(out_v, out_hbm.at[pl.ds(wid * RPW, RPW)])


def kernel(word_table, ent_table, word_ids, entity_ids):
    out_flat = _sc_cosine(word_table, ent_table, word_ids, entity_ids)
    return out_flat.reshape(ROWS // 10, 10)
